# baseline (speedup 1.0000x reference)
import jax
import jax.numpy as jnp
from jax import lax
from jax.experimental import pallas as pl
from jax.experimental.pallas import tpu as pltpu
from jax.experimental.pallas import tpu_sc as plsc

_ROWS = 100000
_KD = 128
_H2 = 64
_BIG = 3.0e38
_IMAX = 2**31 - 1

_CROWS = _ROWS // 2
_CH = 128
_TILE_ROWS = 3136
_NCH = 25
_NBUF = 3

_TBLK = 2000
_TN = _ROWS // _TBLK


def _sc_scan2(cue, keys2):
  mesh = plsc.VectorSubcoreMesh(core_axis_name="c", subcore_axis_name="s",
                                num_cores=2, num_subcores=16)

  def body(cue_hbm, keys_hbm, cd_hbm, ci_hbm,
           b0, b1, b2, cue_v, stage_d, stage_i, alld, alli,
           shd, shi, sem0, sem1, sem2):
    cid = lax.axis_index("c")
    tid = lax.axis_index("s")
    iota16 = lax.iota(jnp.int32, 16)
    bufs = (b0, b1, b2)
    sems = (sem0, sem1, sem2)

    pltpu.sync_copy(cue_hbm.at[0], cue_v.at[pl.ds(0, _KD)])
    cue_v[pl.ds(_KD, 16)] = cue_v[pl.ds(0, 16)]

    cb = cid * _CROWS
    base = cb + tid * _TILE_ROWS
    clamp = cb + _CROWS - _CH
    rows_g = tuple(iota16 + g * 16 for g in range(8))

    def start_chunk(k, b):
      st = jnp.minimum(base + k * _CH, clamp)
      pltpu.make_async_copy(
          keys_hbm.at[pl.ds(st, _CH)], bufs[b], sems[b]).start()

    def wait_chunk(b):
      pltpu.make_async_copy(
          keys_hbm.at[pl.ds(0, _CH)], bufs[b], sems[b]).wait()

    for b in range(_NBUF):
      start_chunk(jnp.int32(b), b)

    def do_chunk(k, b, minv, mini):
      wait_chunk(b)
      st = jnp.minimum(base + k * _CH, clamp)
      accs = tuple(jnp.zeros((16,), jnp.float32) for _ in range(8))

      def s_body(s, carry, b=b):
        jpos = (iota16 + s) & (_KD - 1)
        cv = cue_v[pl.ds(s, 16)]
        accs_t = carry
        out = []
        for g in range(8):
          kv = plsc.load_gather(bufs[b], (rows_g[g], jpos))
          t = kv - cv
          out.append(accs_t[g] + t * t)
        return tuple(out)

      accs = plsc.parallel_loop(0, _KD, unroll=4, carry=accs)(s_body)
      for g in range(8):
        rows = st + g * 16 + iota16
        d = accs[g]
        better = (d < minv) | ((d == minv) & (rows < mini))
        minv = jnp.where(better, d, minv)
        mini = jnp.where(better, rows, mini)
      return minv, mini

    minv = jnp.full((16,), _BIG, jnp.float32)
    mini = jnp.zeros((16,), jnp.int32)

    def outer(c, carry):
      mv, mi = carry
      for b in range(_NBUF):
        k = c * _NBUF + b
        mv, mi = do_chunk(k, b, mv, mi)

        @pl.when(k + _NBUF < _NCH)
        def _(k=k, b=b):
          start_chunk(k + _NBUF, b)
      return mv, mi

    minv, mini = lax.fori_loop(0, (_NCH - 1) // _NBUF, outer, (minv, mini))
    minv, mini = do_chunk(jnp.int32(_NCH - 1), 0, minv, mini)

    stage_d[...] = minv
    stage_i[...] = mini
    pltpu.sync_copy(stage_d, shd.at[pl.ds(tid * 16, 16)])
    pltpu.sync_copy(stage_i, shi.at[pl.ds(tid * 16, 16)])
    plsc.subcore_barrier()

    @pl.when(tid == 0)
    def _():
      pltpu.sync_copy(shd, alld)
      pltpu.sync_copy(shi, alli)
      mv = jnp.full((16,), _BIG, jnp.float32)
      mi = jnp.full((16,), _IMAX, jnp.int32)
      for t in range(16):
        d = alld[pl.ds(t * 16, 16)]
        ii = alli[pl.ds(t * 16, 16)]
        better = (d < mv) | ((d == mv) & (ii < mi))
        mv = jnp.where(better, d, mv)
        mi = jnp.where(better, ii, mi)
      m = jnp.min(mv)
      cand = jnp.where(mv == m, mi, jnp.full((16,), _IMAX, jnp.int32))
      bi = jnp.min(cand)
      stage_d[...] = jnp.full((16,), m, jnp.float32)
      stage_i[...] = jnp.full((16,), bi, jnp.int32)
      pltpu.sync_copy(stage_d, cd_hbm.at[cid])
      pltpu.sync_copy(stage_i, ci_hbm.at[cid])

  f = pl.kernel(
      body,
      out_type=(jax.ShapeDtypeStruct((2, 16), jnp.float32),
                jax.ShapeDtypeStruct((2, 16), jnp.int32)),
      mesh=mesh,
      compiler_params=pltpu.CompilerParams(needs_layout_passes=False),
      scratch_types=[
          pltpu.VMEM((_CH, _KD), jnp.float32),
          pltpu.VMEM((_CH, _KD), jnp.float32),
          pltpu.VMEM((_CH, _KD), jnp.float32),
          pltpu.VMEM((_KD + 16,), jnp.float32),
          pltpu.VMEM((16,), jnp.float32),
          pltpu.VMEM((16,), jnp.int32),
          pltpu.VMEM((256,), jnp.float32),
          pltpu.VMEM((256,), jnp.int32),
          pltpu.VMEM_SHARED((256,), jnp.float32),
          pltpu.VMEM_SHARED((256,), jnp.int32),
          pltpu.SemaphoreType.DMA,
          pltpu.SemaphoreType.DMA,
          pltpu.SemaphoreType.DMA,
      ],
  )
  return f(cue, keys2)


def _tc_scan1_body(keys_r, cue_r, out_r, best_d, best_i):
  i = pl.program_id(0)

  @pl.when(i == 0)
  def _():
    best_d[0] = _BIG
    best_i[0] = 0

  kb = keys_r[...]
  sims = lax.dot_general(kb, cue_r[...], (((1,), (1,)), ((), ())),
                         preferred_element_type=jnp.float32)
  norms = jnp.sum(kb * kb, axis=1, keepdims=True)
  d = norms - 2.0 * sims
  m = jnp.min(d)

  @pl.when(m < best_d[0])
  def _():
    rowid = lax.broadcasted_iota(jnp.int32, (_TBLK, 1), 0)
    li = jnp.min(jnp.where(d == m, rowid, _IMAX))
    best_d[0] = m
    best_i[0] = i * _TBLK + li

  @pl.when(i == _TN - 1)
  def _():
    out_r[0, 0] = best_i[0]


def _tc_scan1(cue, keys1):
  return pl.pallas_call(
      _tc_scan1_body,
      grid=(_TN,),
      in_specs=[
          pl.BlockSpec((_TBLK, _KD), lambda i: (i, 0)),
          pl.BlockSpec((1, _KD), lambda i: (0, 0)),
      ],
      out_specs=pl.BlockSpec(memory_space=pltpu.SMEM),
      out_shape=jax.ShapeDtypeStruct((1, 1), jnp.int32),
      scratch_shapes=[
          pltpu.SMEM((1,), jnp.float32),
          pltpu.SMEM((1,), jnp.int32),
      ],
  )(keys1, cue)


def _sig(x):
  return 1.0 / (1.0 + jnp.exp(-x))


def _dense_body(i1_r, c2d_r, c2i_r, vals1_r, vals2_r,
                obs_r, pa_r, pr_r, h1_r, c1_r, h2_r, c2_r,
                w1_r, be1_r, w2_r, be2_r,
                wih1_r, whh1_r, bi1_r, bh1_r,
                wih2_r, whh2_r, bi2_r, bh2_r,
                aw_r, ab_r, cw_r, cb_r,
                lo_r, vo_r, h1o_r, c1o_r, h2o_r, c2o_r,
                m1_v, m2_v, sem1, sem2):
  i1 = i1_r[0, 0]
  d20 = c2d_r[0, 0]
  d21 = c2d_r[1, 0]
  i20 = c2i_r[0, 0]
  i21 = c2i_r[1, 0]
  use1 = (d21 < d20) | ((d21 == d20) & (i21 < i20))
  i2 = jnp.where(use1, i21, i20)
  cp1 = pltpu.make_async_copy(vals1_r.at[pl.ds(i1, 1)], m1_v, sem1)
  cp2 = pltpu.make_async_copy(vals2_r.at[pl.ds(i2, 1)], m2_v, sem2)
  cp1.start()
  cp2.start()

  def mmT(x, w):
    return lax.dot_general(x, w, (((1,), (1,)), ((), ())),
                           preferred_element_type=jnp.float32)

  obs_v = obs_r[...]
  f1 = jnp.maximum(mmT(obs_v, w1_r[...]) + be1_r[...], 0.0)
  feats = jnp.maximum(mmT(f1, w2_r[...]) + be2_r[...], 0.0)

  wih1 = wih1_r[...]
  h1v = h1_r[...]
  c1v = c1_r[...]
  g = (mmT(feats, wih1[:, 0:128]) + mmT(pr_r[...], wih1[:, 128:129]) +
       mmT(h1v, whh1_r[...]) + bi1_r[...] + bh1_r[...])
  cp1.wait()
  cp2.wait()
  i_g = g[:, 0:128]
  f_g = g[:, 128:256]
  g_g = g[:, 256:384]
  o_g = g[:, 384:512]
  r_g = g[:, 512:640]
  c1n = _sig(f_g) * c1v + _sig(i_g) * jnp.tanh(g_g) + _sig(r_g) * m1_v[...]
  h1n = _sig(o_g) * jnp.tanh(c1n)

  wih2 = wih2_r[...]
  h2v = h2_r[...]
  c2v = c2_r[...]
  g2 = (mmT(h1n, wih2[:, 0:128]) + mmT(feats, wih2[:, 128:256]) +
        mmT(pa_r[...], wih2[:, 256:262]) + mmT(h2v, whh2_r[...]) +
        bi2_r[...] + bh2_r[...])
  i2g = g2[:, 0:64]
  f2g = g2[:, 64:128]
  g2g = g2[:, 128:192]
  o2g = g2[:, 192:256]
  r2g = g2[:, 256:320]
  c2n = _sig(f2g) * c2v + _sig(i2g) * jnp.tanh(g2g) + _sig(r2g) * m2_v[...]
  h2n = _sig(o2g) * jnp.tanh(c2n)

  lo_r[...] = mmT(h2n, aw_r[...]) + ab_r[...]
  vo_r[...] = jnp.sum(h2n * cw_r[...], axis=1, keepdims=True) + cb_r[...]
  h1o_r[...] = h1n
  c1o_r[...] = c1n
  h2o_r[...] = h2n
  c2o_r[...] = c2n


def _dense(idx1, cand2_d, cand2_i, vals1, vals2,
           obs, p_action, p_reward, h1, c1, h2, c2,
           enc_W1, enc_b1, enc_W2, enc_b2,
           Wih1, Whh1, bih1, bhh1, Wih2, Whh2, bih2, bhh2,
           actor_W, actor_b, critic_W, critic_b):
  out_shape = (
      jax.ShapeDtypeStruct((1, 6), jnp.float32),
      jax.ShapeDtypeStruct((1, 1), jnp.float32),
      jax.ShapeDtypeStruct((1, 128), jnp.float32),
      jax.ShapeDtypeStruct((1, 128), jnp.float32),
      jax.ShapeDtypeStruct((1, 64), jnp.float32),
      jax.ShapeDtypeStruct((1, 64), jnp.float32),
  )
  smem = pl.BlockSpec(memory_space=pltpu.SMEM)
  anym = pl.BlockSpec(memory_space=pl.MemorySpace.ANY)
  vmem = pl.BlockSpec(memory_space=pltpu.VMEM)
  in_specs = [smem, smem, smem, anym, anym] + [vmem] * 23
  return pl.pallas_call(
      _dense_body,
      in_specs=in_specs,
      out_shape=out_shape,
      scratch_shapes=[
          pltpu.VMEM((1, _KD), jnp.float32),
          pltpu.VMEM((1, _H2), jnp.float32),
          pltpu.SemaphoreType.DMA,
          pltpu.SemaphoreType.DMA,
      ],
  )(idx1, cand2_d[:, 0:1], cand2_i[:, 0:1], vals1, vals2,
    obs, p_action, p_reward, h1, c1, h2, c2,
    enc_W1, enc_b1.reshape(1, -1), enc_W2, enc_b2.reshape(1, -1),
    Wih1, Whh1, bih1.reshape(1, -1), bhh1.reshape(1, -1),
    Wih2, Whh2, bih2.reshape(1, -1), bhh2.reshape(1, -1),
    actor_W, actor_b.reshape(1, -1), critic_W, critic_b.reshape(1, -1))


def kernel(obs, p_action, p_reward, h1, c1, h2, c2, cue,
           enc_W1, enc_b1, enc_W2, enc_b2, keys1, vals1, keys2, vals2,
           Wih1, Whh1, bih1, bhh1, Wih2, Whh2, bih2, bhh2,
           actor_W, actor_b, critic_W, critic_b):
  cand2_d, cand2_i = _sc_scan2(cue, keys2)
  idx1 = _tc_scan1(cue, keys1)
  lo, vo, h1o, c1o, h2o, c2o = _dense(
      idx1, cand2_d, cand2_i, vals1, vals2,
      obs, p_action, p_reward, h1[0], c1[0], h2[0], c2[0],
      enc_W1, enc_b1, enc_W2, enc_b2,
      Wih1, Whh1, bih1, bhh1, Wih2, Whh2, bih2, bhh2,
      actor_W, actor_b, critic_W, critic_b)
  return (lo[:, None, :], vo[:, None, :], h1o[None], c1o[None],
          h2o[None], c2o[None])
